# Initial kernel scaffold; baseline (speedup 1.0000x reference)
import jax
import jax.numpy as jnp
from jax.experimental import pallas as pl


def kernel(insmi, inlbl, inval, tok_emb, lbl_emb, W1, b1, Wm, bm, Wv, bv, prop_emb):
    raise NotImplementedError("write your pallas kernel here")


# speedup vs baseline: 10.3383x; 10.3383x over previous
import functools

import jax
import jax.numpy as jnp
from jax import lax
from jax.experimental import pallas as pl
from jax.experimental.pallas import tpu as pltpu
from jax.experimental.pallas import tpu_sc as plsc

B = 4096
L = 50
VOCAB = 1000
NLABELS = 1000
EMB = 256
HDIM = 1024
LDIM = 128

NC = 2
NS = 16
NW = NC * NS
LANES = 16
BW = B // NW
RG = 16
NCHUNK = BW // RG

_mesh = plsc.VectorSubcoreMesh(core_axis_name="c", subcore_axis_name="s")


@functools.partial(
    pl.kernel,
    out_type=[
        jax.ShapeDtypeStruct((B * VOCAB,), jnp.float32),
        jax.ShapeDtypeStruct((B, EMB), jnp.float32),
        jax.ShapeDtypeStruct((B, LDIM), jnp.float32),
    ],
    mesh=_mesh,
    compiler_params=pltpu.CompilerParams(needs_layout_passes=False),
    scratch_types=[
        pltpu.VMEM((BW * L,), jnp.int32),
        pltpu.VMEM((BW,), jnp.int32),
        pltpu.VMEM((BW, EMB), jnp.float32),
        pltpu.VMEM((BW, LDIM), jnp.float32),
        pltpu.VMEM((RG * VOCAB,), jnp.float32),
        pltpu.SemaphoreType.DMA,
        pltpu.SemaphoreType.DMA,
    ],
)
def _sc_lookups(insmi_hbm, inlbl_hbm, lbl_emb_hbm, prop_emb_hbm,
                counts_hbm, le_hbm, prop_hbm,
                smi_v, idx_v, lrows_v, prows_v, cnt_v, sem1, sem2):
    wid = lax.axis_index("s") * NC + lax.axis_index("c")
    base = wid * BW

    pltpu.sync_copy(inlbl_hbm.at[pl.ds(base, BW)], idx_v)
    cp_le = pltpu.async_copy(lbl_emb_hbm.at[idx_v], lrows_v, sem1)
    cp_pr = pltpu.async_copy(prop_emb_hbm.at[idx_v], prows_v, sem2)
    pltpu.sync_copy(insmi_hbm.at[pl.ds(base * L, BW * L)], smi_v)

    rows16 = lax.iota(jnp.int32, LANES)
    ones = jnp.full((LANES,), 1.0, jnp.float32)
    zeros = jnp.zeros((LANES,), jnp.float32)

    for g in range(NCHUNK):
        def _zero(j, _):
            cnt_v[pl.ds(j * LANES, LANES)] = zeros
            return 0
        lax.fori_loop(0, RG * VOCAB // LANES, _zero, 0)
        row_sel = g * RG + rows16
        for l in range(L):
            tok = plsc.load_gather(smi_v, [row_sel * L + l])
            plsc.addupdate_scatter(cnt_v, [rows16 * VOCAB + tok], ones)
        pltpu.sync_copy(cnt_v, counts_hbm.at[pl.ds((base + g * RG) * VOCAB, RG * VOCAB)])

    cp_le.wait()
    pltpu.sync_copy(lrows_v, le_hbm.at[pl.ds(base, BW)])
    cp_pr.wait()
    pltpu.sync_copy(prows_v, prop_hbm.at[pl.ds(base, BW)])


BB = 512


def _tc_mlp(cnt_ref, le_ref, tok_ref, w1a_ref, w1b_ref, b1_ref, wz_ref, bz_ref,
            z_ref):
    h_tok = jnp.dot(cnt_ref[...], tok_ref[...],
                    preferred_element_type=jnp.float32) * (1.0 / L)
    pre = (jnp.dot(h_tok, w1a_ref[...], preferred_element_type=jnp.float32)
           + jnp.dot(le_ref[...], w1b_ref[...], preferred_element_type=jnp.float32)
           + b1_ref[...])
    h1 = jnp.tanh(pre)
    z_ref[...] = (jnp.dot(h1, wz_ref[...], preferred_element_type=jnp.float32)
                  + bz_ref[...])


def kernel(insmi, inlbl, inval, tok_emb, lbl_emb, W1, b1, Wm, bm, Wv, bv, prop_emb):
    insmi = insmi.astype(jnp.int32)
    inlbl = inlbl.astype(jnp.int32)

    counts_flat, le, prop = _sc_lookups(insmi.reshape(-1), inlbl, lbl_emb, prop_emb)
    counts = counts_flat.reshape(B, VOCAB)

    W1a = W1[:EMB]
    W1b = W1[EMB:]
    Wz = jnp.concatenate([Wm, Wv], axis=1)
    bz = jnp.concatenate([bm, bv])[None, :]

    z = pl.pallas_call(
        _tc_mlp,
        grid=(B // BB,),
        in_specs=[
            pl.BlockSpec((BB, VOCAB), lambda i: (i, 0)),
            pl.BlockSpec((BB, EMB), lambda i: (i, 0)),
            pl.BlockSpec((VOCAB, EMB), lambda i: (0, 0)),
            pl.BlockSpec((EMB, HDIM), lambda i: (0, 0)),
            pl.BlockSpec((EMB, HDIM), lambda i: (0, 0)),
            pl.BlockSpec((1, HDIM), lambda i: (0, 0)),
            pl.BlockSpec((HDIM, 2 * LDIM), lambda i: (0, 0)),
            pl.BlockSpec((1, 2 * LDIM), lambda i: (0, 0)),
        ],
        out_specs=pl.BlockSpec((BB, 2 * LDIM), lambda i: (i, 0)),
        out_shape=jax.ShapeDtypeStruct((B, 2 * LDIM), jnp.float32),
    )(counts, le, tok_emb, W1a, W1b, b1[None, :], Wz, bz)

    return z[:, :LDIM], z[:, LDIM:], prop


# speedup vs baseline: 17.7105x; 1.7131x over previous
import functools

import jax
import jax.numpy as jnp
from jax import lax
from jax.experimental import pallas as pl
from jax.experimental.pallas import tpu as pltpu
from jax.experimental.pallas import tpu_sc as plsc

B = 4096
L = 50
VOCAB = 1000
VP = 1024
NLABELS = 1000
EMB = 256
HDIM = 1024
LDIM = 128

NC = 2
NS = 16
NW = NC * NS
LANES = 16
BW = B // NW
RG = 16
NCHUNK = BW // RG

_mesh = plsc.VectorSubcoreMesh(core_axis_name="c", subcore_axis_name="s")


@functools.partial(
    pl.kernel,
    out_type=[
        jax.ShapeDtypeStruct((B, VP), jnp.float32),
        jax.ShapeDtypeStruct((B, EMB), jnp.float32),
        jax.ShapeDtypeStruct((B, LDIM), jnp.float32),
    ],
    mesh=_mesh,
    compiler_params=pltpu.CompilerParams(needs_layout_passes=False),
    scratch_types=[
        pltpu.VMEM((BW * L,), jnp.int32),
        pltpu.VMEM((BW,), jnp.int32),
        pltpu.VMEM((BW, EMB), jnp.float32),
        pltpu.VMEM((BW, LDIM), jnp.float32),
        pltpu.VMEM((RG, VP), jnp.float32),
        pltpu.VMEM((RG, VP), jnp.float32),
        pltpu.SemaphoreType.DMA,
        pltpu.SemaphoreType.DMA,
        pltpu.SemaphoreType.DMA,
        pltpu.SemaphoreType.DMA,
    ],
)
def _sc_lookups(insmi_hbm, inlbl_hbm, lbl_emb_hbm, prop_emb_hbm,
                counts_hbm, le_hbm, prop_hbm,
                smi_v, idx_v, lrows_v, prows_v, cnt_a, cnt_b,
                sem_le, sem_pr, sem_ca, sem_cb):
    wid = lax.axis_index("s") * NC + lax.axis_index("c")
    base = wid * BW

    pltpu.sync_copy(inlbl_hbm.at[pl.ds(base, BW)], idx_v)
    cp_le = pltpu.async_copy(lbl_emb_hbm.at[idx_v], lrows_v, sem_le)
    cp_pr = pltpu.async_copy(prop_emb_hbm.at[idx_v], prows_v, sem_pr)
    pltpu.sync_copy(insmi_hbm.at[pl.ds(base * L, BW * L)], smi_v)

    rows16 = lax.iota(jnp.int32, LANES)
    ones = jnp.full((LANES,), 1.0, jnp.float32)
    zeros = jnp.zeros((LANES,), jnp.float32)

    bufs = (cnt_a, cnt_b)
    sems = (sem_ca, sem_cb)

    for buf in bufs:
        flat = RG * VP // LANES
        UNROLL = 16
        def _zero(j, _, buf=buf):
            for u in range(UNROLL):
                buf[(j * UNROLL + u) // (VP // LANES),
                    pl.ds(((j * UNROLL + u) % (VP // LANES)) * LANES, LANES)] = zeros
            return 0
        lax.fori_loop(0, flat // UNROLL, _zero, 0)

    pending = [None, None]
    for g in range(NCHUNK):
        buf = bufs[g % 2]
        sem = sems[g % 2]
        if pending[g % 2] is not None:
            prev_g, cp = pending[g % 2]
            cp.wait()
            prow = (prev_g * RG + rows16) * L
            for l in range(L):
                tok = plsc.load_gather(smi_v, [prow + l])
                plsc.store_scatter(buf, [rows16, tok], zeros)
        row = (g * RG + rows16) * L
        for l in range(L):
            tok = plsc.load_gather(smi_v, [row + l])
            plsc.addupdate_scatter(buf, [rows16, tok], ones)
        pending[g % 2] = (g, pltpu.async_copy(
            buf, counts_hbm.at[pl.ds(base + g * RG, RG)], sem))

    pending[0][1].wait()
    pending[1][1].wait()

    cp_le.wait()
    pltpu.sync_copy(lrows_v, le_hbm.at[pl.ds(base, BW)])
    cp_pr.wait()
    pltpu.sync_copy(prows_v, prop_hbm.at[pl.ds(base, BW)])


BB = 512


def _tc_mlp(cnt_ref, le_ref, tok_ref, w1_ref, b1_ref, wm_ref, bm_ref,
            wv_ref, bv_ref, zm_ref, zlv_ref):
    h_tok = jnp.dot(cnt_ref[:, :VOCAB], tok_ref[...],
                    preferred_element_type=jnp.float32) * (1.0 / L)
    pre = (jnp.dot(h_tok, w1_ref[:EMB, :], preferred_element_type=jnp.float32)
           + jnp.dot(le_ref[...], w1_ref[EMB:, :], preferred_element_type=jnp.float32)
           + b1_ref[...])
    h1 = jnp.tanh(pre)
    zm_ref[...] = (jnp.dot(h1, wm_ref[...], preferred_element_type=jnp.float32)
                   + bm_ref[...])
    zlv_ref[...] = (jnp.dot(h1, wv_ref[...], preferred_element_type=jnp.float32)
                    + bv_ref[...])


def kernel(insmi, inlbl, inval, tok_emb, lbl_emb, W1, b1, Wm, bm, Wv, bv, prop_emb):
    insmi = insmi.astype(jnp.int32)
    inlbl = inlbl.astype(jnp.int32)

    counts, le, prop = _sc_lookups(insmi.reshape(-1), inlbl, lbl_emb, prop_emb)

    zmean, zlogvar = pl.pallas_call(
        _tc_mlp,
        grid=(B // BB,),
        in_specs=[
            pl.BlockSpec((BB, VP), lambda i: (i, 0)),
            pl.BlockSpec((BB, EMB), lambda i: (i, 0)),
            pl.BlockSpec((VOCAB, EMB), lambda i: (0, 0)),
            pl.BlockSpec((2 * EMB, HDIM), lambda i: (0, 0)),
            pl.BlockSpec((1, HDIM), lambda i: (0, 0)),
            pl.BlockSpec((HDIM, LDIM), lambda i: (0, 0)),
            pl.BlockSpec((1, LDIM), lambda i: (0, 0)),
            pl.BlockSpec((HDIM, LDIM), lambda i: (0, 0)),
            pl.BlockSpec((1, LDIM), lambda i: (0, 0)),
        ],
        out_specs=[
            pl.BlockSpec((BB, LDIM), lambda i: (i, 0)),
            pl.BlockSpec((BB, LDIM), lambda i: (i, 0)),
        ],
        out_shape=[
            jax.ShapeDtypeStruct((B, LDIM), jnp.float32),
            jax.ShapeDtypeStruct((B, LDIM), jnp.float32),
        ],
    )(counts, le, tok_emb, W1, b1[None, :], Wm, bm[None, :], Wv, bv[None, :])

    return zmean, zlogvar, prop
